# baseline (speedup 1.0000x reference)
import functools

import jax
import jax.numpy as jnp
from jax import lax
from jax.experimental import pallas as pl
from jax.experimental.pallas import tpu as pltpu
from jax.experimental.pallas import tpu_sc as plsc

D_MODEL = 64
SCALE_F = 8.0
NUM_CORES = 2
NUM_SUBCORES = 16
NUM_WORKERS = NUM_CORES * NUM_SUBCORES
LANES = 16
SBLK = 128
T_STEPS = 200
S_BLOCKS = 4096 // SBLK
N_GROUPS = T_STEPS * S_BLOCKS
G_PER_W = N_GROUPS // NUM_WORKERS


def kernel(token_ids, table):
    batch_shape = token_ids.shape
    n_batch, n_t = batch_shape
    assert (n_batch, n_t) == (4096, 200)
    tokens_t = token_ids.T

    mesh = plsc.VectorSubcoreMesh(core_axis_name="c", subcore_axis_name="s")

    @functools.partial(
        pl.kernel,
        mesh=mesh,
        out_type=jax.ShapeDtypeStruct(
            (T_STEPS, D_MODEL // 8, S_BLOCKS, 8, SBLK), jnp.float32
        ),
        scratch_types=[
            pltpu.VMEM((SBLK,), jnp.int32),
            pltpu.VMEM((SBLK,), jnp.int32),
            pltpu.VMEM((SBLK, D_MODEL), jnp.float32),
            pltpu.VMEM((SBLK, D_MODEL), jnp.float32),
            pltpu.VMEM((D_MODEL // 8, 8, SBLK), jnp.float32),
            pltpu.VMEM((D_MODEL // 8, 8, SBLK), jnp.float32),
            pltpu.SemaphoreType.DMA,
            pltpu.SemaphoreType.DMA,
            pltpu.SemaphoreType.DMA,
            pltpu.SemaphoreType.DMA,
        ],
        compiler_params=pltpu.CompilerParams(
            use_tc_tiling_on_sc=False, needs_layout_passes=False
        ),
    )
    def gather_scale(table_hbm, tok_hbm, out_hbm, idx0, idx1, rows0, rows1,
                     buf0, buf1, sem_g0, sem_g1, sem_o0, sem_o1):
        wid = lax.axis_index("s") * NUM_CORES + lax.axis_index("c")
        g_base = wid * G_PER_W
        idx_v = (idx0, idx1)
        rows_v = (rows0, rows1)
        buf_v = (buf0, buf1)
        sem_g = (sem_g0, sem_g1)
        sem_o = (sem_o0, sem_o1)
        iota16 = lax.iota(jnp.int32, LANES)
        srow = [iota16 + LANES * k for k in range(SBLK // LANES)]

        def t_sb(j):
            g = g_base + j
            return g >> 5, g & (S_BLOCKS - 1)

        def start_gather(j, b):
            t, sb = t_sb(j)
            pltpu.sync_copy(tok_hbm.at[t, pl.ds(sb * SBLK, SBLK)], idx_v[b])
            pltpu.async_copy(table_hbm.at[idx_v[b]], rows_v[b], sem_g[b])

        start_gather(0, 0)
        start_gather(1, 1)

        @pl.loop(0, G_PER_W, step=2)
        def _(jj):
            for b in range(2):
                j = jj + b
                t, sb = t_sb(j)
                pltpu.make_async_copy(
                    table_hbm.at[idx_v[b]], rows_v[b], sem_g[b]
                ).wait()

                @pl.when(jj >= 2)
                def _():
                    for a in range(D_MODEL // 8):
                        pltpu.make_async_copy(
                            buf_v[b].at[a], out_hbm.at[t, a, sb], sem_o[b]
                        ).wait()

                @pl.loop(0, D_MODEL)
                def _(d):
                    dcol = jnp.full((LANES,), d, jnp.int32)
                    a = d >> 3
                    r = d & 7
                    for k in range(SBLK // LANES):
                        vals = plsc.load_gather(rows_v[b], [srow[k], dcol])
                        buf_v[b].at[a, r, pl.ds(k * LANES, LANES)][...] = (
                            vals * SCALE_F
                        )

                for a in range(D_MODEL // 8):
                    pltpu.async_copy(
                        buf_v[b].at[a], out_hbm.at[t, a, sb], sem_o[b]
                    )

                @pl.when(jj + 2 < G_PER_W)
                def _():
                    start_gather(j + 2, b)

        for j in (G_PER_W - 2, G_PER_W - 1):
            b = j % 2
            t, sb = t_sb(j)
            for a in range(D_MODEL // 8):
                pltpu.make_async_copy(
                    buf_v[b].at[a], out_hbm.at[t, a, sb], sem_o[b]
                ).wait()

    out = gather_scale(table, tokens_t)
    return out.transpose(2, 4, 0, 1, 3).reshape(*batch_shape, D_MODEL)


# speedup vs baseline: 1.3135x; 1.3135x over previous
import functools

import jax
import jax.numpy as jnp
from jax import lax
from jax.experimental import pallas as pl
from jax.experimental.pallas import tpu as pltpu
from jax.experimental.pallas import tpu_sc as plsc

D_MODEL = 64
D_PAD = 128
SCALE_F = 8.0
NUM_CORES = 2
NUM_SUBCORES = 16
NUM_WORKERS = NUM_CORES * NUM_SUBCORES
LANES = 16
VOCAB = 1000000
PACK_CHUNK = 400
N_PACK_CHUNKS = VOCAB // PACK_CHUNK
CHUNK = 800


def _pack_scale(table):
    mesh = plsc.VectorSubcoreMesh(core_axis_name="c", subcore_axis_name="s")

    @functools.partial(
        pl.kernel,
        mesh=mesh,
        out_type=jax.ShapeDtypeStruct((VOCAB // 2, D_PAD), jnp.float32),
        scratch_types=[
            pltpu.VMEM((PACK_CHUNK, D_MODEL), jnp.float32),
            pltpu.VMEM((PACK_CHUNK // 2, D_PAD), jnp.float32),
            pltpu.SemaphoreType.DMA,
        ],
        compiler_params=pltpu.CompilerParams(use_tc_tiling_on_sc=True),
    )
    def pack(table_hbm, out_hbm, vin, vout, sem):
        wid = lax.axis_index("s") * NUM_CORES + lax.axis_index("c")
        n_mine = (N_PACK_CHUNKS + NUM_WORKERS - 1) // NUM_WORKERS

        @pl.loop(0, n_mine)
        def _(i):
            ci = wid + NUM_WORKERS * i

            @pl.when(ci < N_PACK_CHUNKS)
            def _():
                base = pl.multiple_of(ci * PACK_CHUNK, PACK_CHUNK)
                pltpu.sync_copy(table_hbm.at[pl.ds(base, PACK_CHUNK)], vin)

                @pl.loop(0, PACK_CHUNK, step=2)
                def _(r):
                    for h in range(2):
                        for c in range(0, D_MODEL, LANES):
                            vout.at[r // 2,
                                    pl.ds(h * D_MODEL + c, LANES)][...] = (
                                vin.at[r + h, pl.ds(c, LANES)][...] * SCALE_F
                            )

                pltpu.sync_copy(
                    vout,
                    out_hbm.at[pl.ds(pl.multiple_of(base // 2,
                                                    PACK_CHUNK // 2),
                                     PACK_CHUNK // 2)],
                )

    return pack(table)


def _gather(scaled_flat, idx, num_ids):
    per_worker = num_ids // NUM_WORKERS
    n_chunks = per_worker // CHUNK
    assert n_chunks * CHUNK == per_worker and n_chunks >= 2

    mesh = plsc.VectorSubcoreMesh(core_axis_name="c", subcore_axis_name="s")

    @functools.partial(
        pl.kernel,
        mesh=mesh,
        out_type=jax.ShapeDtypeStruct((num_ids, D_PAD), jnp.float32),
        scratch_types=[
            pltpu.VMEM((CHUNK,), jnp.int32),
            pltpu.VMEM((CHUNK,), jnp.int32),
            pltpu.VMEM((CHUNK, D_MODEL), jnp.float32),
            pltpu.VMEM((CHUNK, D_MODEL), jnp.float32),
            pltpu.SemaphoreType.DMA,
            pltpu.SemaphoreType.DMA,
            pltpu.SemaphoreType.DMA,
            pltpu.SemaphoreType.DMA,
        ],
        compiler_params=pltpu.CompilerParams(use_tc_tiling_on_sc=False),
    )
    def gather(table_hbm, idx_hbm, out_hbm, idx0, idx1, rows0, rows1,
               sem_g0, sem_g1, sem_o0, sem_o1):
        wid = lax.axis_index("s") * NUM_CORES + lax.axis_index("c")
        base0 = wid * per_worker
        idx_v = (idx0, idx1)
        rows_v = (rows0, rows1)
        sem_g = (sem_g0, sem_g1)
        sem_o = (sem_o0, sem_o1)

        def start_gather(j, b):
            base = base0 + j * CHUNK
            pltpu.sync_copy(idx_hbm.at[pl.ds(base, CHUNK)], idx_v[b])
            return pltpu.async_copy(table_hbm.at[idx_v[b]], rows_v[b],
                                    sem_g[b])

        gathers = [None, None]
        outs = [None, None]
        gathers[0] = start_gather(0, 0)

        for j in range(n_chunks):
            b = j % 2
            if j + 1 < n_chunks:
                gathers[(j + 1) % 2] = start_gather(j + 1, (j + 1) % 2)
            gathers[b].wait()
            if outs[b] is not None:
                outs[b].wait()
            base = base0 + j * CHUNK
            outs[b] = pltpu.async_copy(
                rows_v[b],
                out_hbm.at[pl.ds(base, CHUNK), pl.ds(0, D_MODEL)],
                sem_o[b],
            )

        outs[(n_chunks - 2) % 2].wait()
        outs[(n_chunks - 1) % 2].wait()

    return gather(scaled_flat, idx)


def kernel(token_ids, table):
    batch_shape = token_ids.shape
    idx = token_ids.reshape(-1)
    num_ids = idx.shape[0]

    packed = _pack_scale(table)
    scaled = packed.reshape(VOCAB, D_MODEL)
    out = _gather(scaled, idx, num_ids)
    return out[:, :D_MODEL].reshape(*batch_shape, D_MODEL)


# speedup vs baseline: 2.1609x; 1.6451x over previous
import functools

import jax
import jax.numpy as jnp
from jax import lax
from jax.experimental import pallas as pl
from jax.experimental.pallas import tpu as pltpu
from jax.experimental.pallas import tpu_sc as plsc

D_MODEL = 64
D_PAD = 128
SCALE_F = 8.0
NUM_CORES = 2
NUM_SUBCORES = 16
NUM_WORKERS = NUM_CORES * NUM_SUBCORES
LANES = 16
CHUNK = 800


def kernel(token_ids, table):
    batch_shape = token_ids.shape
    idx = token_ids.reshape(-1)
    num_ids = idx.shape[0]
    per_worker = num_ids // NUM_WORKERS
    n_chunks = per_worker // CHUNK
    assert per_worker * NUM_WORKERS == num_ids
    assert n_chunks * CHUNK == per_worker
    assert n_chunks >= 2

    mesh = plsc.VectorSubcoreMesh(core_axis_name="c", subcore_axis_name="s")

    @functools.partial(
        pl.kernel,
        mesh=mesh,
        out_type=jax.ShapeDtypeStruct((num_ids, D_PAD), jnp.float32),
        scratch_types=[
            pltpu.VMEM((CHUNK,), jnp.int32),
            pltpu.VMEM((CHUNK,), jnp.int32),
            pltpu.VMEM((CHUNK, D_MODEL), jnp.float32),
            pltpu.VMEM((CHUNK, D_MODEL), jnp.float32),
            pltpu.SemaphoreType.DMA,
            pltpu.SemaphoreType.DMA,
            pltpu.SemaphoreType.DMA,
            pltpu.SemaphoreType.DMA,
        ],
        compiler_params=pltpu.CompilerParams(use_tc_tiling_on_sc=False),
    )
    def gather_scale(table_hbm, idx_hbm, out_hbm, idx0, idx1, rows0, rows1,
                     sem_g0, sem_g1, sem_o0, sem_o1):
        wid = lax.axis_index("s") * NUM_CORES + lax.axis_index("c")
        base0 = wid * per_worker
        idx_v = (idx0, idx1)
        rows_v = (rows0, rows1)
        sem_g = (sem_g0, sem_g1)
        sem_o = (sem_o0, sem_o1)

        def start_gather(j, b):
            base = base0 + j * CHUNK
            pltpu.sync_copy(idx_hbm.at[pl.ds(base, CHUNK)], idx_v[b])
            return pltpu.async_copy(table_hbm.at[idx_v[b]], rows_v[b],
                                    sem_g[b])

        gathers = [None, None]
        outs = [None, None]
        gathers[0] = start_gather(0, 0)

        for j in range(n_chunks):
            b = j % 2
            if j + 1 < n_chunks:
                gathers[(j + 1) % 2] = start_gather(j + 1, (j + 1) % 2)
            gathers[b].wait()
            if outs[b] is not None:
                outs[b].wait()

            @pl.loop(0, CHUNK)
            def _(r):
                for c in range(0, D_MODEL, LANES):
                    sl = (r, pl.ds(c, LANES))
                    rows_v[b].at[sl][...] = rows_v[b].at[sl][...] * SCALE_F

            base = base0 + j * CHUNK
            outs[b] = pltpu.async_copy(
                rows_v[b],
                out_hbm.at[pl.ds(base, CHUNK), pl.ds(0, D_MODEL)],
                sem_o[b],
            )

        outs[(n_chunks - 2) % 2].wait()
        outs[(n_chunks - 1) % 2].wait()

    out = gather_scale(table, idx)
    return out[:, :D_MODEL].reshape(*batch_shape, D_MODEL)
